# baseline (speedup 1.0000x reference)
import numpy as np
import jax
import jax.numpy as jnp
from jax.experimental import pallas as pl
from jax.experimental.pallas import tpu as pltpu
from jax.experimental.pallas import tpu_sc as plsc

_NUM_HEADS = 4
_HEAD = 32768
_BATCH = 128
_RB = 16


def _gumbel_table() -> np.ndarray:
    n = np.arange(_BATCH * _HEAD, dtype=np.uint64)
    x0 = (n >> np.uint64(32)).astype(np.uint32)
    x1 = (n & np.uint64(0xFFFFFFFF)).astype(np.uint32)
    ks0 = np.uint32(0)
    ks1 = np.uint32(42)
    ks2 = np.uint32(ks0 ^ ks1 ^ np.uint32(0x1BD11BDA))
    ks = (ks0, ks1, ks2)
    rot = ((13, 15, 26, 6), (17, 29, 16, 24))
    x0 = (x0 + ks0).astype(np.uint32)
    x1 = (x1 + ks1).astype(np.uint32)
    for g in range(5):
        for r in rot[g % 2]:
            x0 = (x0 + x1).astype(np.uint32)
            x1 = ((x1 << np.uint32(r)) | (x1 >> np.uint32(32 - r))).astype(np.uint32)
            x1 = (x1 ^ x0).astype(np.uint32)
        x0 = (x0 + ks[(g + 1) % 3]).astype(np.uint32)
        x1 = (x1 + ks[(g + 2) % 3] + np.uint32(g + 1)).astype(np.uint32)
    bits = (x0 ^ x1).astype(np.uint32)
    tiny = np.float32(np.finfo(np.float32).tiny)
    f = ((bits >> np.uint32(9)) | np.uint32(0x3F800000)).view(np.float32)
    u = f - np.float32(1.0)
    u = np.maximum(tiny, u * (np.float32(1.0) - tiny) + tiny)
    gum = (-np.log(-np.log(u))).astype(np.float32)
    return gum.reshape(_BATCH, _HEAD)


_GUMBEL = _gumbel_table()


_CW = 512
_NC = _HEAD // _CW


def _body(x0_ref, x1_ref, x2_ref, x3_ref, g_ref, o_ref):
    x_refs = (x0_ref, x1_ref, x2_ref, x3_ref)
    neg_inf = jnp.full((_RB, _CW), -jnp.inf, jnp.float32)
    zeros = jnp.zeros((_RB, _CW), jnp.int32)

    def step(c, carry):
        ms, idxs = carry
        off = c * _CW
        gc = g_ref[:, pl.ds(off, _CW)]
        new_ms, new_idxs = [], []
        for h in range(_NUM_HEADS):
            v = x_refs[h][:, pl.ds(off, _CW)] + gc
            upd = v > ms[h]
            new_ms.append(jnp.where(upd, v, ms[h]))
            new_idxs.append(jnp.where(upd, c, idxs[h]))
        return tuple(new_ms), tuple(new_idxs)

    ms, idxs = jax.lax.fori_loop(
        0, _NC, step, ((neg_inf,) * _NUM_HEADS, (zeros,) * _NUM_HEADS))

    lane = jax.lax.broadcasted_iota(jnp.int32, (_RB, _CW), 1)
    for h in range(_NUM_HEADS):
        m = jnp.max(ms[h], axis=-1, keepdims=True)
        gidx = idxs[h] * _CW + lane
        idx = jnp.min(jnp.where(ms[h] == m, gidx, jnp.int32(_HEAD)), axis=-1)
        o_ref[0, h, :] = idx


def _kernel_tc(x):
    g = jnp.asarray(_GUMBEL)
    grid = (_BATCH // _RB,)

    def _head_spec(h):
        return pl.BlockSpec((_RB, _HEAD), lambda i, _h=h: (i, _h))

    out = pl.pallas_call(
        _body,
        grid=grid,
        in_specs=[_head_spec(0), _head_spec(1), _head_spec(2), _head_spec(3),
                  pl.BlockSpec((_RB, _HEAD), lambda i: (i, 0))],
        out_specs=pl.BlockSpec((1, _NUM_HEADS, _RB), lambda i: (i, 0, 0)),
        out_shape=jax.ShapeDtypeStruct((_BATCH // _RB, _NUM_HEADS, _RB), jnp.int32),
    )(x, x, x, x, g)
    return out.transpose(1, 0, 2).reshape(_NUM_HEADS * _BATCH)


_CH = 16384
_NCH = _HEAD // _CH
_VPC = _CH // 16
_ROWS_PER_W = 4


def _xlane(v, perm):
    return jax.lax.gather(
        v, perm[:, None],
        jax.lax.GatherDimensionNumbers(
            offset_dims=(), collapsed_slice_dims=(0,), start_index_map=(0,)),
        (1,), mode=jax.lax.GatherScatterMode.PROMISE_IN_BOUNDS)


def _sc_body(x_hbm, g_hbm, o_hbm, xb0, xb1, xb2, xb3, gb, res):
    wid = jax.lax.axis_index("s") * 2 + jax.lax.axis_index("c")
    xbufs = (xb0, xb1, xb2, xb3)
    iota = jax.lax.iota(jnp.int32, 16)
    acc = jnp.zeros((16,), jnp.int32)
    for ri in range(_ROWS_PER_W):
        row = wid * _ROWS_PER_W + ri
        ms = (jnp.full((16,), -jnp.inf, jnp.float32),) * _NUM_HEADS
        idxs = (jnp.zeros((16,), jnp.int32),) * _NUM_HEADS
        for c in range(_NCH):
            pltpu.sync_copy(g_hbm.at[row, pl.ds(c * _CH, _CH)], gb)
            for h in range(_NUM_HEADS):
                pltpu.sync_copy(
                    x_hbm.at[row, pl.ds(h * _HEAD + c * _CH, _CH)], xbufs[h])

            def step(i, carry, _c=c):
                cms, cidxs = carry
                gv = gb[pl.ds(i * 16, 16)]
                gi = _c * _VPC + i
                nms, nidxs = [], []
                for h in range(_NUM_HEADS):
                    v = xbufs[h][pl.ds(i * 16, 16)] + gv
                    upd = v > cms[h]
                    nms.append(jnp.where(upd, v, cms[h]))
                    nidxs.append(jnp.where(upd, gi, cidxs[h]))
                return tuple(nms), tuple(nidxs)

            ms, idxs = jax.lax.fori_loop(0, _VPC, step, (ms, idxs))
        for h in range(_NUM_HEADS):
            m = ms[h]
            for sh in (1, 2, 4, 8):
                m = jnp.maximum(m, _xlane(m, iota ^ sh))
            gidx = idxs[h] * 16 + iota
            cand = jnp.where(ms[h] == m, gidx, jnp.int32(_HEAD))
            for sh in (1, 2, 4, 8):
                cand = jnp.minimum(cand, _xlane(cand, iota ^ sh))
            acc = jnp.where(iota == (ri * _NUM_HEADS + h), cand, acc)
    res[...] = acc
    pltpu.sync_copy(res, o_hbm.at[pl.ds(wid * 16, 16)])


def _kernel_sc(x):
    g = jnp.asarray(_GUMBEL)
    fn = pl.kernel(
        _sc_body,
        out_type=jax.ShapeDtypeStruct((_BATCH * _NUM_HEADS,), jnp.int32),
        mesh=plsc.VectorSubcoreMesh(core_axis_name="c", subcore_axis_name="s"),
        scratch_types=[pltpu.VMEM((_CH,), jnp.float32)] * 5
        + [pltpu.VMEM((16,), jnp.int32)],
    )
    out = fn(x, g)
    return out.reshape(_BATCH, _NUM_HEADS).T.reshape(_NUM_HEADS * _BATCH)


def kernel(x):
    return _kernel_sc(x)


# speedup vs baseline: 1.9040x; 1.9040x over previous
import functools

import numpy as np
import jax
import jax.numpy as jnp
from jax.experimental import pallas as pl
from jax.experimental.pallas import tpu as pltpu
from jax.experimental.pallas import tpu_sc as plsc

_NUM_HEADS = 4
_HEAD = 32768
_BATCH = 128
_RB = 16


def _gumbel_table() -> np.ndarray:
    n = np.arange(_BATCH * _HEAD, dtype=np.uint64)
    x0 = (n >> np.uint64(32)).astype(np.uint32)
    x1 = (n & np.uint64(0xFFFFFFFF)).astype(np.uint32)
    ks0 = np.uint32(0)
    ks1 = np.uint32(42)
    ks2 = np.uint32(ks0 ^ ks1 ^ np.uint32(0x1BD11BDA))
    ks = (ks0, ks1, ks2)
    rot = ((13, 15, 26, 6), (17, 29, 16, 24))
    x0 = (x0 + ks0).astype(np.uint32)
    x1 = (x1 + ks1).astype(np.uint32)
    for g in range(5):
        for r in rot[g % 2]:
            x0 = (x0 + x1).astype(np.uint32)
            x1 = ((x1 << np.uint32(r)) | (x1 >> np.uint32(32 - r))).astype(np.uint32)
            x1 = (x1 ^ x0).astype(np.uint32)
        x0 = (x0 + ks[(g + 1) % 3]).astype(np.uint32)
        x1 = (x1 + ks[(g + 2) % 3] + np.uint32(g + 1)).astype(np.uint32)
    bits = (x0 ^ x1).astype(np.uint32)
    tiny = np.float32(np.finfo(np.float32).tiny)
    f = ((bits >> np.uint32(9)) | np.uint32(0x3F800000)).view(np.float32)
    u = f - np.float32(1.0)
    u = np.maximum(tiny, u * (np.float32(1.0) - tiny) + tiny)
    gum = (-np.log(-np.log(u))).astype(np.float32)
    return gum.reshape(_BATCH, _HEAD)


_GUMBEL = _gumbel_table()


_CW = 512
_NC = _HEAD // _CW


def _body(x0_ref, x1_ref, x2_ref, x3_ref, g_ref, o_ref):
    x_refs = (x0_ref, x1_ref, x2_ref, x3_ref)
    neg_inf = jnp.full((_RB, _CW), -jnp.inf, jnp.float32)
    zeros = jnp.zeros((_RB, _CW), jnp.int32)

    def step(c, carry):
        ms, idxs = carry
        off = c * _CW
        gc = g_ref[:, pl.ds(off, _CW)]
        new_ms, new_idxs = [], []
        for h in range(_NUM_HEADS):
            v = x_refs[h][:, pl.ds(off, _CW)] + gc
            upd = v > ms[h]
            new_ms.append(jnp.where(upd, v, ms[h]))
            new_idxs.append(jnp.where(upd, c, idxs[h]))
        return tuple(new_ms), tuple(new_idxs)

    ms, idxs = jax.lax.fori_loop(
        0, _NC, step, ((neg_inf,) * _NUM_HEADS, (zeros,) * _NUM_HEADS))

    lane = jax.lax.broadcasted_iota(jnp.int32, (_RB, _CW), 1)
    for h in range(_NUM_HEADS):
        m = jnp.max(ms[h], axis=-1, keepdims=True)
        gidx = idxs[h] * _CW + lane
        idx = jnp.min(jnp.where(ms[h] == m, gidx, jnp.int32(_HEAD)), axis=-1)
        o_ref[0, h, :] = idx


def _kernel_tc(x):
    g = jnp.asarray(_GUMBEL)
    grid = (_BATCH // _RB,)

    def _head_spec(h):
        return pl.BlockSpec((_RB, _HEAD), lambda i, _h=h: (i, _h))

    out = pl.pallas_call(
        _body,
        grid=grid,
        in_specs=[_head_spec(0), _head_spec(1), _head_spec(2), _head_spec(3),
                  pl.BlockSpec((_RB, _HEAD), lambda i: (i, 0))],
        out_specs=pl.BlockSpec((1, _NUM_HEADS, _RB), lambda i: (i, 0, 0)),
        out_shape=jax.ShapeDtypeStruct((_BATCH // _RB, _NUM_HEADS, _RB), jnp.int32),
    )(x, x, x, x, g)
    return out.transpose(1, 0, 2).reshape(_NUM_HEADS * _BATCH)


_CH = 16384
_NCH = _HEAD // _CH
_VPC = _CH // 16
_ROWS_PER_W = 4


def _xlane(v, perm):
    return jax.lax.gather(
        v, perm[:, None],
        jax.lax.GatherDimensionNumbers(
            offset_dims=(), collapsed_slice_dims=(0,), start_index_map=(0,)),
        (1,), mode=jax.lax.GatherScatterMode.PROMISE_IN_BOUNDS)


def _sc_body(x_hbm, g_hbm, o_hbm, xb0, xb1, xb2, xb3, gb, res,
             rows_per_w=_ROWS_PER_W, row_base=0):
    wid = jax.lax.axis_index("s") * 2 + jax.lax.axis_index("c")
    xbufs = (xb0, xb1, xb2, xb3)
    iota = jax.lax.iota(jnp.int32, 16)
    acc = jnp.zeros((16,), jnp.int32)
    for ri in range(rows_per_w):
        row = row_base + wid * rows_per_w + ri
        ms = (jnp.full((16,), -jnp.inf, jnp.float32),) * _NUM_HEADS
        idxs = (jnp.zeros((16,), jnp.int32),) * _NUM_HEADS
        for c in range(_NCH):
            pltpu.sync_copy(g_hbm.at[row, pl.ds(c * _CH, _CH)], gb)
            for h in range(_NUM_HEADS):
                pltpu.sync_copy(
                    x_hbm.at[row, pl.ds(h * _HEAD + c * _CH, _CH)], xbufs[h])

            def step(i, carry, _c=c):
                cms, cidxs = carry
                gv = gb[pl.ds(i * 16, 16)]
                gi = _c * _VPC + i
                nms, nidxs = [], []
                for h in range(_NUM_HEADS):
                    v = xbufs[h][pl.ds(i * 16, 16)] + gv
                    upd = v > cms[h]
                    nms.append(jnp.where(upd, v, cms[h]))
                    nidxs.append(jnp.where(upd, gi, cidxs[h]))
                return tuple(nms), tuple(nidxs)

            ms, idxs = jax.lax.fori_loop(0, _VPC, step, (ms, idxs))
        for h in range(_NUM_HEADS):
            m = ms[h]
            for sh in (1, 2, 4, 8):
                m = jnp.maximum(m, _xlane(m, iota ^ sh))
            gidx = idxs[h] * 16 + iota
            cand = jnp.where(ms[h] == m, gidx, jnp.int32(_HEAD))
            for sh in (1, 2, 4, 8):
                cand = jnp.minimum(cand, _xlane(cand, iota ^ sh))
            acc = jnp.where(iota == (ri * _NUM_HEADS + h), cand, acc)
    res[...] = acc
    pltpu.sync_copy(res, o_hbm.at[wid])


_SC_SCRATCH = [pltpu.VMEM((_CH,), jnp.float32)] * 5 + [pltpu.VMEM((16,), jnp.int32)]
_SC_MESH = dict(core_axis_name="c", subcore_axis_name="s")


def _kernel_sc(x):
    g = jnp.asarray(_GUMBEL)
    fn = pl.kernel(
        _sc_body,
        out_type=jax.ShapeDtypeStruct((32, 16), jnp.int32),
        mesh=plsc.VectorSubcoreMesh(**_SC_MESH),
        scratch_types=_SC_SCRATCH,
    )
    out = fn(x, g)
    return out.reshape(_BATCH, _NUM_HEADS).T.reshape(_NUM_HEADS * _BATCH)


_TC_ROWS = 96


def _kernel_hybrid(x):
    g = jnp.asarray(_GUMBEL)

    def _head_spec(h):
        return pl.BlockSpec((_RB, _HEAD), lambda i, _h=h: (i, _h))

    tc_out = pl.pallas_call(
        _body,
        grid=(_TC_ROWS // _RB,),
        in_specs=[_head_spec(0), _head_spec(1), _head_spec(2), _head_spec(3),
                  pl.BlockSpec((_RB, _HEAD), lambda i: (i, 0))],
        out_specs=pl.BlockSpec((1, _NUM_HEADS, _RB), lambda i: (i, 0, 0)),
        out_shape=jax.ShapeDtypeStruct((_TC_ROWS // _RB, _NUM_HEADS, _RB),
                                       jnp.int32),
    )(x, x, x, x, g)

    sc_rpw = (_BATCH - _TC_ROWS) // 32
    sc_fn = pl.kernel(
        functools.partial(_sc_body, rows_per_w=sc_rpw, row_base=_TC_ROWS),
        out_type=jax.ShapeDtypeStruct((32, 16), jnp.int32),
        mesh=plsc.VectorSubcoreMesh(**_SC_MESH),
        scratch_types=_SC_SCRATCH,
    )
    sc_out = sc_fn(x, g)

    tc_part = tc_out.transpose(1, 0, 2).reshape(_NUM_HEADS, _TC_ROWS)
    sc_part = sc_out[:, : sc_rpw * _NUM_HEADS].reshape(
        _BATCH - _TC_ROWS, _NUM_HEADS).T
    return jnp.concatenate([tc_part, sc_part], axis=1).reshape(
        _NUM_HEADS * _BATCH)


def kernel(x):
    return _kernel_hybrid(x)


# speedup vs baseline: 3.3511x; 1.7600x over previous
import functools

import numpy as np
import jax
import jax.numpy as jnp
from jax.experimental import pallas as pl
from jax.experimental.pallas import tpu as pltpu
from jax.experimental.pallas import tpu_sc as plsc

_NUM_HEADS = 4
_HEAD = 32768
_BATCH = 128
_RB = 16


def _gumbel_table() -> np.ndarray:
    n = np.arange(_BATCH * _HEAD, dtype=np.uint64)
    x0 = (n >> np.uint64(32)).astype(np.uint32)
    x1 = (n & np.uint64(0xFFFFFFFF)).astype(np.uint32)
    ks0 = np.uint32(0)
    ks1 = np.uint32(42)
    ks2 = np.uint32(ks0 ^ ks1 ^ np.uint32(0x1BD11BDA))
    ks = (ks0, ks1, ks2)
    rot = ((13, 15, 26, 6), (17, 29, 16, 24))
    x0 = (x0 + ks0).astype(np.uint32)
    x1 = (x1 + ks1).astype(np.uint32)
    for g in range(5):
        for r in rot[g % 2]:
            x0 = (x0 + x1).astype(np.uint32)
            x1 = ((x1 << np.uint32(r)) | (x1 >> np.uint32(32 - r))).astype(np.uint32)
            x1 = (x1 ^ x0).astype(np.uint32)
        x0 = (x0 + ks[(g + 1) % 3]).astype(np.uint32)
        x1 = (x1 + ks[(g + 2) % 3] + np.uint32(g + 1)).astype(np.uint32)
    bits = (x0 ^ x1).astype(np.uint32)
    tiny = np.float32(np.finfo(np.float32).tiny)
    f = ((bits >> np.uint32(9)) | np.uint32(0x3F800000)).view(np.float32)
    u = f - np.float32(1.0)
    u = np.maximum(tiny, u * (np.float32(1.0) - tiny) + tiny)
    gum = (-np.log(-np.log(u))).astype(np.float32)
    return gum.reshape(_BATCH, _HEAD)


_GUMBEL = _gumbel_table()


_CW = 512
_NC = _HEAD // _CW


def _body(x0_ref, x1_ref, x2_ref, x3_ref, g_ref, o_ref):
    x_refs = (x0_ref, x1_ref, x2_ref, x3_ref)
    neg_inf = jnp.full((_RB, _CW), -jnp.inf, jnp.float32)
    zeros = jnp.zeros((_RB, _CW), jnp.int32)

    def step(c, carry):
        ms, idxs = carry
        off = c * _CW
        gc = g_ref[:, pl.ds(off, _CW)]
        new_ms, new_idxs = [], []
        for h in range(_NUM_HEADS):
            v = x_refs[h][:, pl.ds(off, _CW)] + gc
            upd = v > ms[h]
            new_ms.append(jnp.where(upd, v, ms[h]))
            new_idxs.append(jnp.where(upd, c, idxs[h]))
        return tuple(new_ms), tuple(new_idxs)

    ms, idxs = jax.lax.fori_loop(
        0, _NC, step, ((neg_inf,) * _NUM_HEADS, (zeros,) * _NUM_HEADS))

    lane = jax.lax.broadcasted_iota(jnp.int32, (_RB, _CW), 1)
    for h in range(_NUM_HEADS):
        m = jnp.max(ms[h], axis=-1, keepdims=True)
        gidx = idxs[h] * _CW + lane
        idx = jnp.min(jnp.where(ms[h] == m, gidx, jnp.int32(_HEAD)), axis=-1)
        o_ref[0, h, :] = idx


def _kernel_tc(x):
    g = jnp.asarray(_GUMBEL)
    grid = (_BATCH // _RB,)

    def _head_spec(h):
        return pl.BlockSpec((_RB, _HEAD), lambda i, _h=h: (i, _h))

    out = pl.pallas_call(
        _body,
        grid=grid,
        in_specs=[_head_spec(0), _head_spec(1), _head_spec(2), _head_spec(3),
                  pl.BlockSpec((_RB, _HEAD), lambda i: (i, 0))],
        out_specs=pl.BlockSpec((1, _NUM_HEADS, _RB), lambda i: (i, 0, 0)),
        out_shape=jax.ShapeDtypeStruct((_BATCH // _RB, _NUM_HEADS, _RB),
                                       jnp.int32),
    )(x, x, x, x, g)
    return out.transpose(1, 0, 2).reshape(_NUM_HEADS * _BATCH)


_CH = 16384
_NCH = _HEAD // _CH
_VPC = _CH // 16
_ROWS_PER_W = 4


def _xlane(v, perm):
    return jax.lax.gather(
        v, perm[:, None],
        jax.lax.GatherDimensionNumbers(
            offset_dims=(), collapsed_slice_dims=(0,), start_index_map=(0,)),
        (1,), mode=jax.lax.GatherScatterMode.PROMISE_IN_BOUNDS)


def _sc_body(x_hbm, g_hbm, o_hbm, xb0, xb1, xb2, xb3, gb, res,
             rows_per_w=_ROWS_PER_W, row_base=0):
    wid = jax.lax.axis_index("s") * 2 + jax.lax.axis_index("c")
    xbufs = (xb0, xb1, xb2, xb3)
    iota = jax.lax.iota(jnp.int32, 16)
    acc = jnp.zeros((16,), jnp.int32)
    for ri in range(rows_per_w):
        row = row_base + wid * rows_per_w + ri
        ms = (jnp.full((16,), -jnp.inf, jnp.float32),) * _NUM_HEADS
        idxs = (jnp.zeros((16,), jnp.int32),) * _NUM_HEADS
        for c in range(_NCH):
            pltpu.sync_copy(g_hbm.at[row, pl.ds(c * _CH, _CH)], gb)
            for h in range(_NUM_HEADS):
                pltpu.sync_copy(
                    x_hbm.at[row, pl.ds(h * _HEAD + c * _CH, _CH)], xbufs[h])

            def step(i, carry, _c=c):
                cms, cidxs = carry
                gv = gb[pl.ds(i * 16, 16)]
                gi = _c * _VPC + i
                nms, nidxs = [], []
                for h in range(_NUM_HEADS):
                    v = xbufs[h][pl.ds(i * 16, 16)] + gv
                    upd = v > cms[h]
                    nms.append(jnp.where(upd, v, cms[h]))
                    nidxs.append(jnp.where(upd, gi, cidxs[h]))
                return tuple(nms), tuple(nidxs)

            ms, idxs = jax.lax.fori_loop(0, _VPC, step, (ms, idxs))
        for h in range(_NUM_HEADS):
            m = ms[h]
            for sh in (1, 2, 4, 8):
                m = jnp.maximum(m, _xlane(m, iota ^ sh))
            gidx = idxs[h] * 16 + iota
            cand = jnp.where(ms[h] == m, gidx, jnp.int32(_HEAD))
            for sh in (1, 2, 4, 8):
                cand = jnp.minimum(cand, _xlane(cand, iota ^ sh))
            acc = jnp.where(iota == (ri * _NUM_HEADS + h), cand, acc)
    res[...] = acc
    pltpu.sync_copy(res, o_hbm.at[wid])


_SC_SCRATCH = [pltpu.VMEM((_CH,), jnp.float32)] * 5 + [pltpu.VMEM((16,), jnp.int32)]
_SC_MESH = dict(core_axis_name="c", subcore_axis_name="s")


def _kernel_sc(x):
    g = jnp.asarray(_GUMBEL)
    fn = pl.kernel(
        _sc_body,
        out_type=jax.ShapeDtypeStruct((32, 16), jnp.int32),
        mesh=plsc.VectorSubcoreMesh(**_SC_MESH),
        scratch_types=_SC_SCRATCH,
    )
    out = fn(x, g)
    return out.reshape(_BATCH, _NUM_HEADS).T.reshape(_NUM_HEADS * _BATCH)


_TC_ROWS = 96


def _kernel_hybrid(x):
    g = jnp.asarray(_GUMBEL)

    def _head_spec(h):
        return pl.BlockSpec((_RB, _HEAD), lambda i, _h=h: (i, _h))

    sc_rpw = (_BATCH - _TC_ROWS) // 32
    sc_fn = pl.kernel(
        functools.partial(_sc_body, rows_per_w=sc_rpw, row_base=_TC_ROWS),
        out_type=jax.ShapeDtypeStruct((32, 16), jnp.int32),
        mesh=plsc.VectorSubcoreMesh(**_SC_MESH),
        scratch_types=_SC_SCRATCH,
    )
    sc_out = sc_fn(x, g)

    tc_out = pl.pallas_call(
        _body,
        grid=(_TC_ROWS // _RB,),
        in_specs=[_head_spec(0), _head_spec(1), _head_spec(2), _head_spec(3),
                  pl.BlockSpec((_RB, _HEAD), lambda i: (i, 0))],
        out_specs=pl.BlockSpec((1, _NUM_HEADS, _RB), lambda i: (i, 0, 0)),
        out_shape=jax.ShapeDtypeStruct((_TC_ROWS // _RB, _NUM_HEADS, _RB),
                                       jnp.int32),
    )(x, x, x, x, g)

    tc_part = tc_out.transpose(1, 0, 2).reshape(_NUM_HEADS, _TC_ROWS)
    sc_part = sc_out[:, : sc_rpw * _NUM_HEADS].reshape(
        _BATCH - _TC_ROWS, _NUM_HEADS).T
    return jnp.concatenate([tc_part, sc_part], axis=1).reshape(
        _NUM_HEADS * _BATCH)


def kernel(x):
    return _kernel_tc(x)


# speedup vs baseline: 3.4194x; 1.0204x over previous
import functools

import numpy as np
import jax
import jax.numpy as jnp
from jax.experimental import pallas as pl
from jax.experimental.pallas import tpu as pltpu
from jax.experimental.pallas import tpu_sc as plsc

_NUM_HEADS = 4
_HEAD = 32768
_BATCH = 128
_RB = 16


def _gumbel_table() -> np.ndarray:
    n = np.arange(_BATCH * _HEAD, dtype=np.uint64)
    x0 = (n >> np.uint64(32)).astype(np.uint32)
    x1 = (n & np.uint64(0xFFFFFFFF)).astype(np.uint32)
    ks0 = np.uint32(0)
    ks1 = np.uint32(42)
    ks2 = np.uint32(ks0 ^ ks1 ^ np.uint32(0x1BD11BDA))
    ks = (ks0, ks1, ks2)
    rot = ((13, 15, 26, 6), (17, 29, 16, 24))
    x0 = (x0 + ks0).astype(np.uint32)
    x1 = (x1 + ks1).astype(np.uint32)
    for g in range(5):
        for r in rot[g % 2]:
            x0 = (x0 + x1).astype(np.uint32)
            x1 = ((x1 << np.uint32(r)) | (x1 >> np.uint32(32 - r))).astype(np.uint32)
            x1 = (x1 ^ x0).astype(np.uint32)
        x0 = (x0 + ks[(g + 1) % 3]).astype(np.uint32)
        x1 = (x1 + ks[(g + 2) % 3] + np.uint32(g + 1)).astype(np.uint32)
    bits = (x0 ^ x1).astype(np.uint32)
    tiny = np.float32(np.finfo(np.float32).tiny)
    f = ((bits >> np.uint32(9)) | np.uint32(0x3F800000)).view(np.float32)
    u = f - np.float32(1.0)
    u = np.maximum(tiny, u * (np.float32(1.0) - tiny) + tiny)
    gum = (-np.log(-np.log(u))).astype(np.float32)
    return gum.reshape(_BATCH, _HEAD)


_GUMBEL = _gumbel_table()


_CW = 512
_NC = _HEAD // _CW


def _body(x0_ref, x1_ref, x2_ref, x3_ref, g_ref, o_ref):
    x_refs = (x0_ref, x1_ref, x2_ref, x3_ref)
    neg_inf = jnp.full((_RB, _CW), -jnp.inf, jnp.float32)
    zeros = jnp.zeros((_RB, _CW), jnp.int32)

    def step(c, carry):
        ms, idxs = carry
        off = c * _CW
        gc = g_ref[:, pl.ds(off, _CW)]
        new_ms, new_idxs = [], []
        for h in range(_NUM_HEADS):
            v = x_refs[h][:, pl.ds(off, _CW)] + gc
            upd = v > ms[h]
            new_ms.append(jnp.where(upd, v, ms[h]))
            new_idxs.append(jnp.where(upd, c, idxs[h]))
        return tuple(new_ms), tuple(new_idxs)

    ms, idxs = jax.lax.fori_loop(
        0, _NC, step, ((neg_inf,) * _NUM_HEADS, (zeros,) * _NUM_HEADS))

    lane = jax.lax.broadcasted_iota(jnp.int32, (_RB, _CW), 1)
    for h in range(_NUM_HEADS):
        m = jnp.max(ms[h], axis=-1, keepdims=True)
        gidx = idxs[h] * _CW + lane
        idx = jnp.min(jnp.where(ms[h] == m, gidx, jnp.int32(_HEAD)), axis=-1)
        o_ref[0, h, :] = idx


def _kernel_tc(x):
    g = jnp.asarray(_GUMBEL)
    grid = (_BATCH // _RB,)

    def _head_spec(h):
        return pl.BlockSpec((_RB, _HEAD), lambda i, _h=h: (i, _h))

    out = pl.pallas_call(
        _body,
        grid=grid,
        in_specs=[_head_spec(0), _head_spec(1), _head_spec(2), _head_spec(3),
                  pl.BlockSpec((_RB, _HEAD), lambda i: (i, 0))],
        out_specs=pl.BlockSpec((1, _NUM_HEADS, _RB), lambda i: (i, 0, 0)),
        out_shape=jax.ShapeDtypeStruct((_BATCH // _RB, _NUM_HEADS, _RB),
                                       jnp.int32),
    )(x, x, x, x, g)
    return out.transpose(1, 0, 2).reshape(_NUM_HEADS * _BATCH)


_CH = 16384
_NCH = _HEAD // _CH
_VPC = _CH // 16
_ROWS_PER_W = 4


def _xlane(v, perm):
    return jax.lax.gather(
        v, perm[:, None],
        jax.lax.GatherDimensionNumbers(
            offset_dims=(), collapsed_slice_dims=(0,), start_index_map=(0,)),
        (1,), mode=jax.lax.GatherScatterMode.PROMISE_IN_BOUNDS)


def _sc_body(x_hbm, g_hbm, o_hbm, xb0, xb1, xb2, xb3, gb, res,
             rows_per_w=_ROWS_PER_W, row_base=0):
    wid = jax.lax.axis_index("s") * 2 + jax.lax.axis_index("c")
    xbufs = (xb0, xb1, xb2, xb3)
    iota = jax.lax.iota(jnp.int32, 16)
    acc = jnp.zeros((16,), jnp.int32)
    for ri in range(rows_per_w):
        row = row_base + wid * rows_per_w + ri
        ms = (jnp.full((16,), -jnp.inf, jnp.float32),) * _NUM_HEADS
        idxs = (jnp.zeros((16,), jnp.int32),) * _NUM_HEADS
        for c in range(_NCH):
            pltpu.sync_copy(g_hbm.at[row, pl.ds(c * _CH, _CH)], gb)
            for h in range(_NUM_HEADS):
                pltpu.sync_copy(
                    x_hbm.at[row, pl.ds(h * _HEAD + c * _CH, _CH)], xbufs[h])

            def step(i, carry, _c=c):
                cms, cidxs = carry
                gv = gb[pl.ds(i * 16, 16)]
                gi = _c * _VPC + i
                nms, nidxs = [], []
                for h in range(_NUM_HEADS):
                    v = xbufs[h][pl.ds(i * 16, 16)] + gv
                    upd = v > cms[h]
                    nms.append(jnp.where(upd, v, cms[h]))
                    nidxs.append(jnp.where(upd, gi, cidxs[h]))
                return tuple(nms), tuple(nidxs)

            ms, idxs = jax.lax.fori_loop(0, _VPC, step, (ms, idxs))
        for h in range(_NUM_HEADS):
            m = ms[h]
            for sh in (1, 2, 4, 8):
                m = jnp.maximum(m, _xlane(m, iota ^ sh))
            gidx = idxs[h] * 16 + iota
            cand = jnp.where(ms[h] == m, gidx, jnp.int32(_HEAD))
            for sh in (1, 2, 4, 8):
                cand = jnp.minimum(cand, _xlane(cand, iota ^ sh))
            acc = jnp.where(iota == (ri * _NUM_HEADS + h), cand, acc)
    res[...] = acc
    pltpu.sync_copy(res, o_hbm.at[wid])


_SC_SCRATCH = [pltpu.VMEM((_CH,), jnp.float32)] * 5 + [pltpu.VMEM((16,), jnp.int32)]
_SC_MESH = dict(core_axis_name="c", subcore_axis_name="s")


def _kernel_sc(x):
    g = jnp.asarray(_GUMBEL)
    fn = pl.kernel(
        _sc_body,
        out_type=jax.ShapeDtypeStruct((32, 16), jnp.int32),
        mesh=plsc.VectorSubcoreMesh(**_SC_MESH),
        scratch_types=_SC_SCRATCH,
    )
    out = fn(x, g)
    return out.reshape(_BATCH, _NUM_HEADS).T.reshape(_NUM_HEADS * _BATCH)


_TC_ROWS = 96


def _kernel_hybrid(x):
    g = jnp.asarray(_GUMBEL)

    def _head_spec(h):
        return pl.BlockSpec((_RB, _HEAD), lambda i, _h=h: (i, _h))

    sc_rpw = (_BATCH - _TC_ROWS) // 32
    sc_fn = pl.kernel(
        functools.partial(_sc_body, rows_per_w=sc_rpw, row_base=_TC_ROWS),
        out_type=jax.ShapeDtypeStruct((32, 16), jnp.int32),
        mesh=plsc.VectorSubcoreMesh(**_SC_MESH),
        scratch_types=_SC_SCRATCH,
    )
    sc_out = sc_fn(x, g)

    tc_out = pl.pallas_call(
        _body,
        grid=(_TC_ROWS // _RB,),
        in_specs=[_head_spec(0), _head_spec(1), _head_spec(2), _head_spec(3),
                  pl.BlockSpec((_RB, _HEAD), lambda i: (i, 0))],
        out_specs=pl.BlockSpec((1, _NUM_HEADS, _RB), lambda i: (i, 0, 0)),
        out_shape=jax.ShapeDtypeStruct((_TC_ROWS // _RB, _NUM_HEADS, _RB),
                                       jnp.int32),
    )(x, x, x, x, g)

    tc_part = tc_out.transpose(1, 0, 2).reshape(_NUM_HEADS, _TC_ROWS)
    sc_part = sc_out[:, : sc_rpw * _NUM_HEADS].reshape(
        _BATCH - _TC_ROWS, _NUM_HEADS).T
    return jnp.concatenate([tc_part, sc_part], axis=1).reshape(
        _NUM_HEADS * _BATCH)


_HH = _HEAD // 2


def _body8(xa0, xb0_, xa1, xb1_, xa2, xb2_, xa3, xb3_, ga, gb_, o_ref):
    xs = ((xa0, xb0_), (xa1, xb1_), (xa2, xb2_), (xa3, xb3_))
    gs = (ga, gb_)
    neg_inf = jnp.full((_RB, _CW), -jnp.inf, jnp.float32)
    zeros = jnp.zeros((_RB, _CW), jnp.int32)

    def make_step(half):
        def step(c, carry):
            ms, idxs = carry
            off = c * _CW
            gc = gs[half][:, pl.ds(off, _CW)]
            base = half * (_HH // _CW)
            new_ms, new_idxs = [], []
            for h in range(_NUM_HEADS):
                v = xs[h][half][:, pl.ds(off, _CW)] + gc
                upd = v > ms[h]
                new_ms.append(jnp.where(upd, v, ms[h]))
                new_idxs.append(jnp.where(upd, base + c, idxs[h]))
            return tuple(new_ms), tuple(new_idxs)
        return step

    carry = ((neg_inf,) * _NUM_HEADS, (zeros,) * _NUM_HEADS)
    carry = jax.lax.fori_loop(0, _HH // _CW, make_step(0), carry)
    ms, idxs = jax.lax.fori_loop(0, _HH // _CW, make_step(1), carry)

    lane = jax.lax.broadcasted_iota(jnp.int32, (_RB, _CW), 1)
    for h in range(_NUM_HEADS):
        m = jnp.max(ms[h], axis=-1, keepdims=True)
        gidx = idxs[h] * _CW + lane
        idx = jnp.min(jnp.where(ms[h] == m, gidx, jnp.int32(_HEAD)), axis=-1)
        o_ref[0, h, :] = idx


def _kernel_tc8(x):
    g = jnp.asarray(_GUMBEL)

    def _spec(colblk):
        return pl.BlockSpec((_RB, _HH), lambda i, _cb=colblk: (i, _cb))

    specs = []
    for h in range(_NUM_HEADS):
        specs += [_spec(2 * h), _spec(2 * h + 1)]
    gspecs = [pl.BlockSpec((_RB, _HH), lambda i: (i, 0)),
              pl.BlockSpec((_RB, _HH), lambda i: (i, 1))]
    out = pl.pallas_call(
        _body8,
        grid=(_BATCH // _RB,),
        in_specs=specs + gspecs,
        out_specs=pl.BlockSpec((1, _NUM_HEADS, _RB), lambda i: (i, 0, 0)),
        out_shape=jax.ShapeDtypeStruct((_BATCH // _RB, _NUM_HEADS, _RB),
                                       jnp.int32),
    )(x, x, x, x, x, x, x, x, g, g)
    return out.transpose(1, 0, 2).reshape(_NUM_HEADS * _BATCH)


def kernel(x):
    return _kernel_tc8(x)
